# baseline (speedup 1.0000x reference)
import functools

import jax
import jax.numpy as jnp
import numpy as np
from jax import lax
from jax.experimental import pallas as pl
from jax.experimental.pallas import tpu as pltpu
from jax.experimental.pallas import tpu_sc as plsc

BATCH = 16384
NUM_FIELDS = 26
EMBED_DIM = 64
NUM_INTERACTIONS = 100
LANES = 16
P_PAD = 112
NCHUNK_P = P_PAD // LANES
S_ROW = NUM_INTERACTIONS
NB = 32
ROW_W = NUM_FIELDS * EMBED_DIM


def _make_sc_kernel(num_workers):
    rows_pw = BATCH // num_workers
    nchunks = rows_pw // NB
    mesh = plsc.VectorSubcoreMesh(core_axis_name="c", subcore_axis_name="s")

    @functools.partial(
        pl.kernel,
        mesh=mesh,
        compiler_params=pltpu.CompilerParams(needs_layout_passes=False),
        out_type=jax.ShapeDtypeStruct((BATCH,), jnp.float32),
        scratch_types=[
            pltpu.VMEM((NB * ROW_W,), jnp.float32),
            pltpu.VMEM((NB * ROW_W,), jnp.float32),
            pltpu.VMEM((NB * S_ROW,), jnp.float32),
            pltpu.VMEM((NB * S_ROW,), jnp.float32),
            pltpu.VMEM((P_PAD,), jnp.int32),
            pltpu.VMEM((P_PAD,), jnp.int32),
            pltpu.VMEM((EMBED_DIM,), jnp.float32),
            pltpu.VMEM((LANES,), jnp.float32),
            pltpu.VMEM((rows_pw,), jnp.float32),
            pltpu.SemaphoreType.DMA,
            pltpu.SemaphoreType.DMA,
            pltpu.SemaphoreType.DMA,
            pltpu.SemaphoreType.DMA,
        ],
    )
    def sc_kernel(emb_hbm, sc_hbm, off1_hbm, off2_hbm, w_hbm, b_hbm, out_hbm,
                  emb_v0, emb_v1, sc_v0, sc_v1, off1_v, off2_v, w_v, b_v, out_v,
                  sem_e0, sem_e1, sem_s0, sem_s1):
        emb_v = (emb_v0, emb_v1)
        sc_v = (sc_v0, sc_v1)
        wid = lax.axis_index("s") * 2 + lax.axis_index("c")
        row0 = wid * rows_pw

        pltpu.sync_copy(off1_hbm, off1_v)
        pltpu.sync_copy(off2_hbm, off2_v)
        pltpu.sync_copy(w_hbm, w_v)
        pltpu.sync_copy(b_hbm, b_v)
        bvec = b_v[...]

        sem_e = (sem_e0, sem_e1)
        sem_s = (sem_s0, sem_s1)

        def issue(chunk, slot):
            base = (row0 + chunk * NB)
            e = pltpu.async_copy(
                emb_hbm.at[pl.ds(base * ROW_W, NB * ROW_W)], emb_v[slot],
                sem_e[slot])
            s = pltpu.async_copy(
                sc_hbm.at[pl.ds(base * S_ROW, NB * S_ROW)], sc_v[slot],
                sem_s[slot])
            del e, s

        def wait(chunk, slot):
            base = (row0 + chunk * NB)
            pltpu.make_async_copy(
                emb_hbm.at[pl.ds(base * ROW_W, NB * ROW_W)], emb_v[slot],
                sem_e[slot]).wait()
            pltpu.make_async_copy(
                sc_hbm.at[pl.ds(base * S_ROW, NB * S_ROW)], sc_v[slot],
                sem_s[slot]).wait()

        issue(0, 0)
        issue(1, 1)

        def do_row_pair(r, chunk, slot):
            ebase = r * ROW_W
            b1 = [off1_v[pl.ds(pc * LANES, LANES)] + ebase
                  for pc in range(NCHUNK_P)]
            b2 = [off2_v[pl.ds(pc * LANES, LANES)] + ebase
                  for pc in range(NCHUNK_P)]
            eref = emb_v[slot]

            lane = lax.iota(jnp.int32, LANES)

            def dstep(d, accs):
                dl = (lane + d) & (EMBED_DIM - 1)
                wv = plsc.load_gather(w_v, [dl])
                newa, newb = [], []
                for pc in range(NCHUNK_P):
                    i1a = b1[pc] + dl
                    i2a = b2[pc] + dl
                    g1a = plsc.load_gather(eref, [i1a])
                    g2a = plsc.load_gather(eref, [i2a])
                    newa.append(accs[pc] + g1a * g2a * wv)
                    g1b = plsc.load_gather(eref, [i1a + ROW_W])
                    g2b = plsc.load_gather(eref, [i2a + ROW_W])
                    newb.append(accs[NCHUNK_P + pc] + g1b * g2b * wv)
                return tuple(newa + newb)

            accs = plsc.parallel_loop(
                0, EMBED_DIM, unroll=1,
                carry=tuple(jnp.zeros((LANES,), jnp.float32)
                            for _ in range(2 * NCHUNK_P)))(dstep)

            for half in range(2):
                sbase = (r + half) * S_ROW
                t = bvec
                for pc in range(NCHUNK_P):
                    off = pc * LANES if pc < 6 else S_ROW - LANES
                    sv = sc_v[slot][pl.ds(sbase + off, LANES)]
                    if pc == 6:
                        sv = jnp.where(lane >= 12, sv, 0.0)
                    t = t + accs[half * NCHUNK_P + pc] * sv
                val = jnp.sum(t)
                plsc.store_scatter(
                    out_v,
                    [jnp.full((LANES,), 0, jnp.int32) + (chunk * NB + r + half)],
                    jnp.zeros((LANES,), jnp.float32) + val,
                    mask=lane == 0)

        def do_chunk(chunk, slot):
            wait(chunk, slot)

            def rbody(rr, _):
                do_row_pair(rr * 2, chunk, slot)
                return 0

            lax.fori_loop(0, NB // 2, rbody, 0)

            @pl.when(chunk + 2 < nchunks)
            def _():
                issue(chunk + 2, slot)

        def gbody(g2, _):
            do_chunk(g2 * 2, 0)
            do_chunk(g2 * 2 + 1, 1)
            return 0

        lax.fori_loop(0, nchunks // 2, gbody, 0)
        pltpu.sync_copy(out_v, out_hbm.at[pl.ds(row0, rows_pw)])

    return sc_kernel


@jax.jit
def kernel(embeddings, selected_pairs, interaction_scores, W, b):
    emb_flat = embeddings.reshape(-1)
    order = np.concatenate([np.arange(96), np.arange(84, 100)]).astype(np.int32)
    off1 = selected_pairs[order, 0] * EMBED_DIM
    off2 = selected_pairs[order, 1] * EMBED_DIM
    sc_flat = interaction_scores.reshape(-1)
    w_flat = W.reshape(-1)
    b_pad = jnp.pad(b, (0, LANES - 1))

    sc_fn = _make_sc_kernel(32)
    out = sc_fn(emb_flat, sc_flat, off1.astype(jnp.int32),
                off2.astype(jnp.int32), w_flat, b_pad)
    return out.reshape(BATCH, 1)


# speedup vs baseline: 1.3714x; 1.3714x over previous
import functools

import jax
import jax.numpy as jnp
import numpy as np
from jax import lax
from jax.experimental import pallas as pl
from jax.experimental.pallas import tpu as pltpu
from jax.experimental.pallas import tpu_sc as plsc

BATCH = 16384
NUM_FIELDS = 26
EMBED_DIM = 64
NUM_INTERACTIONS = 100
LANES = 16
P_PAD = 112
NCHUNK_P = P_PAD // LANES
S_ROW = NUM_INTERACTIONS
NB = 16
ROW_W = NUM_FIELDS * EMBED_DIM


def _make_sc_kernel(num_workers):
    rows_pw = BATCH // num_workers
    nchunks = rows_pw // NB
    mesh = plsc.VectorSubcoreMesh(core_axis_name="c", subcore_axis_name="s")

    @functools.partial(
        pl.kernel,
        mesh=mesh,
        compiler_params=pltpu.CompilerParams(needs_layout_passes=False),
        out_type=jax.ShapeDtypeStruct((BATCH,), jnp.float32),
        scratch_types=[
            pltpu.VMEM((NB * ROW_W,), jnp.float32),
            pltpu.VMEM((NB * ROW_W,), jnp.float32),
            pltpu.VMEM((NB * S_ROW,), jnp.float32),
            pltpu.VMEM((NB * S_ROW,), jnp.float32),
            pltpu.VMEM((P_PAD,), jnp.int32),
            pltpu.VMEM((P_PAD,), jnp.int32),
            pltpu.VMEM((EMBED_DIM,), jnp.float32),
            pltpu.VMEM((LANES,), jnp.float32),
            pltpu.VMEM((rows_pw,), jnp.float32),
            pltpu.SemaphoreType.DMA,
            pltpu.SemaphoreType.DMA,
            pltpu.SemaphoreType.DMA,
            pltpu.SemaphoreType.DMA,
        ],
    )
    def sc_kernel(emb_hbm, sc_hbm, off1_hbm, off2_hbm, w_hbm, b_hbm, out_hbm,
                  emb_v0, emb_v1, sc_v0, sc_v1, off1_v, off2_v, w_v, b_v, out_v,
                  sem_e0, sem_e1, sem_s0, sem_s1):
        emb_v = (emb_v0, emb_v1)
        sc_v = (sc_v0, sc_v1)
        wid = lax.axis_index("s") * 2 + lax.axis_index("c")
        row0 = wid * rows_pw

        pltpu.sync_copy(off1_hbm, off1_v)
        pltpu.sync_copy(off2_hbm, off2_v)
        pltpu.sync_copy(w_hbm, w_v)
        pltpu.sync_copy(b_hbm, b_v)
        bvec = b_v[...]

        sem_e = (sem_e0, sem_e1)
        sem_s = (sem_s0, sem_s1)

        def issue(chunk, slot):
            base = (row0 + chunk * NB)
            e = pltpu.async_copy(
                emb_hbm.at[pl.ds(base * ROW_W, NB * ROW_W)], emb_v[slot],
                sem_e[slot])
            s = pltpu.async_copy(
                sc_hbm.at[pl.ds(base * S_ROW, NB * S_ROW)], sc_v[slot],
                sem_s[slot])
            del e, s

        def wait(chunk, slot):
            base = (row0 + chunk * NB)
            pltpu.make_async_copy(
                emb_hbm.at[pl.ds(base * ROW_W, NB * ROW_W)], emb_v[slot],
                sem_e[slot]).wait()
            pltpu.make_async_copy(
                sc_hbm.at[pl.ds(base * S_ROW, NB * S_ROW)], sc_v[slot],
                sem_s[slot]).wait()

        issue(0, 0)
        issue(1, 1)

        def do_row(r, chunk, slot):
            ebase = r * ROW_W
            sbase = r * S_ROW
            b1 = [off1_v[pl.ds(pc * LANES, LANES)] + ebase
                  for pc in range(NCHUNK_P)]
            b2 = [off2_v[pl.ds(pc * LANES, LANES)] + ebase
                  for pc in range(NCHUNK_P)]
            eref = emb_v[slot]

            lane = lax.iota(jnp.int32, LANES)

            def dstep(d, accs):
                dl = (lane + d) & (EMBED_DIM - 1)
                wv = plsc.load_gather(w_v, [dl])
                new = []
                for pc in range(NCHUNK_P):
                    g1 = plsc.load_gather(eref, [b1[pc] + dl])
                    g2 = plsc.load_gather(eref, [b2[pc] + dl])
                    new.append(accs[pc] + g1 * g2 * wv)
                return tuple(new)

            accs = plsc.parallel_loop(
                0, EMBED_DIM, unroll=1,
                carry=tuple(jnp.zeros((LANES,), jnp.float32)
                            for _ in range(NCHUNK_P)))(dstep)
            t = bvec
            for pc in range(NCHUNK_P):
                off = pc * LANES if pc < 6 else S_ROW - LANES
                sv = sc_v[slot][pl.ds(sbase + off, LANES)]
                if pc == 6:
                    sv = jnp.where(lane >= 12, sv, 0.0)
                t = t + accs[pc] * sv
            val = jnp.sum(t)
            lane = lax.iota(jnp.int32, LANES)
            plsc.store_scatter(out_v, [jnp.full((LANES,), 0, jnp.int32) + (chunk * NB + r)],
                               jnp.zeros((LANES,), jnp.float32) + val,
                               mask=lane == 0)

        def do_chunk(chunk, slot):
            wait(chunk, slot)

            def rbody(r, _):
                do_row(r, chunk, slot)
                return 0

            lax.fori_loop(0, NB, rbody, 0)

            @pl.when(chunk + 2 < nchunks)
            def _():
                issue(chunk + 2, slot)

        def gbody(g2, _):
            do_chunk(g2 * 2, 0)
            do_chunk(g2 * 2 + 1, 1)
            return 0

        lax.fori_loop(0, nchunks // 2, gbody, 0)
        pltpu.sync_copy(out_v, out_hbm.at[pl.ds(row0, rows_pw)])

    return sc_kernel


@jax.jit
def kernel(embeddings, selected_pairs, interaction_scores, W, b):
    emb_flat = embeddings.reshape(-1)
    order = np.concatenate([np.arange(96), np.arange(84, 100)]).astype(np.int32)
    off1 = selected_pairs[order, 0] * EMBED_DIM
    off2 = selected_pairs[order, 1] * EMBED_DIM
    sc_flat = interaction_scores.reshape(-1)
    w_flat = W.reshape(-1)
    b_pad = jnp.pad(b, (0, LANES - 1))

    sc_fn = _make_sc_kernel(32)
    out = sc_fn(emb_flat, sc_flat, off1.astype(jnp.int32),
                off2.astype(jnp.int32), w_flat, b_pad)
    return out.reshape(BATCH, 1)
